# initial kernel scaffold (unmeasured)
import jax
import jax.numpy as jnp
from jax import lax
from jax.experimental import pallas as pl
from jax.experimental.pallas import tpu as pltpu

N_DEV = 4


def kernel(x, w_mat):
    M, _ = x.shape
    _, N = w_mat.shape
    MC = M // N_DEV

    x = x.astype(jnp.bfloat16)
    w = w_mat.astype(jnp.bfloat16)

    def body(x_ref, w_ref, out_ref, y_ref, rs_recv, rs_send,
             rs_recv_sems, rs_send_sems, ag_send_sems, ag_recv_sems):
        my = lax.axis_index("i")
        left = lax.rem(my + (N_DEV - 1), N_DEV)
        right = lax.rem(my + 1, N_DEV)

        barrier_sem = pltpu.get_barrier_semaphore()
        for nbr in (left, right):
            pl.semaphore_signal(
                barrier_sem, inc=1, device_id=(nbr,),
                device_id_type=pl.DeviceIdType.MESH,
            )
        pl.semaphore_wait(barrier_sem, 2)

        def partial_chunk(c):
            xs = x_ref[pl.ds(c * MC, MC), :]
            return lax.dot_general(
                xs, w_ref[...], (((1,), (0,)), ((), ())),
                preferred_element_type=jnp.float32,
            )

        def rs_rdma(send_slot, hop):
            return pltpu.make_async_remote_copy(
                src_ref=rs_send.at[send_slot],
                dst_ref=rs_recv.at[hop],
                send_sem=rs_send_sems.at[send_slot],
                recv_sem=rs_recv_sems.at[hop],
                device_id=(right,),
                device_id_type=pl.DeviceIdType.MESH,
            )

        p = partial_chunk(my)
        rs_send[0, :, :] = p.astype(jnp.bfloat16)
        rdma0 = rs_rdma(0, 0)
        rdma0.start()
        p = partial_chunk(lax.rem(my + (N_DEV - 1), N_DEV))
        rdma0.wait_recv()
        rs_send[1, :, :] = (p + rs_recv[0].astype(jnp.float32)).astype(jnp.bfloat16)
        rdma1 = rs_rdma(1, 1)
        rdma1.start()
        p = partial_chunk(lax.rem(my + (N_DEV - 2), N_DEV))
        rdma0.wait_send()
        rdma1.wait_recv()
        rs_send[0, :, :] = (p + rs_recv[1].astype(jnp.float32)).astype(jnp.bfloat16)
        rdma2 = rs_rdma(0, 2)
        rdma2.start()
        p = partial_chunk(lax.rem(my + (N_DEV - 3), N_DEV))
        rdma2.wait_recv()
        c_own = lax.rem(my + 1, N_DEV)
        y_ref[pl.ds(c_own * MC, MC), :] = (
            p + rs_recv[2].astype(jnp.float32)
        ).astype(jnp.bfloat16)
        rdma1.wait_send()
        rdma2.wait_send()

        for h in range(N_DEV - 1):
            c_s = lax.rem(my + 1 + h, N_DEV)
            rows = pl.ds(c_s * MC, MC)
            ag = pltpu.make_async_remote_copy(
                src_ref=y_ref.at[rows, :],
                dst_ref=y_ref.at[rows, :],
                send_sem=ag_send_sems.at[h],
                recv_sem=ag_recv_sems.at[h],
                device_id=(left,),
                device_id_type=pl.DeviceIdType.MESH,
            )
            ag.start()
            ag.wait()

        amax = jnp.maximum(jnp.max(y_ref[...]).astype(jnp.float32), 1e-30)
        scale = amax / 448.0
        inv = 448.0 / amax
        for c in range(N_DEV):
            r = jnp.maximum(
                y_ref[pl.ds(c * MC, MC), :], 0
            ).astype(jnp.float32)
            q = jnp.minimum(r * inv, 448.0).astype(jnp.float8_e4m3fn)
            out_ref[pl.ds(c * MC, MC), :] = q.astype(jnp.float32) * scale

    return pl.pallas_call(
        body,
        out_shape=jax.ShapeDtypeStruct((M, N), jnp.float32),
        in_specs=[
            pl.BlockSpec(memory_space=pltpu.VMEM),
            pl.BlockSpec(memory_space=pltpu.VMEM),
        ],
        out_specs=pl.BlockSpec(memory_space=pltpu.VMEM),
        scratch_shapes=[
            pltpu.VMEM((M, N), jnp.bfloat16),
            pltpu.VMEM((N_DEV - 1, MC, N), jnp.bfloat16),
            pltpu.VMEM((2, MC, N), jnp.bfloat16),
            pltpu.SemaphoreType.DMA((N_DEV - 1,)),
            pltpu.SemaphoreType.DMA((2,)),
            pltpu.SemaphoreType.DMA((N_DEV - 1,)),
            pltpu.SemaphoreType.DMA((N_DEV - 1,)),
        ],
        compiler_params=pltpu.CompilerParams(collective_id=0),
    )(x, w)


# baseline (device time: 360165 ns/iter reference)
import jax
import jax.numpy as jnp
from jax import lax
from jax.experimental import pallas as pl
from jax.experimental.pallas import tpu as pltpu

N_DEV = 4


def kernel(x, w_mat):
    M, _ = x.shape
    _, N = w_mat.shape
    MC = M // N_DEV
    SC = 512
    n_sub = M // SC

    x = x.astype(jnp.bfloat16)
    w = w_mat.astype(jnp.bfloat16)

    def body(x_ref, w_ref, out_ref, y_ref, rs_recv, rs_send, stage,
             rs_recv_sems, rs_send_sem, ag_send_sems, ag_recv_sems,
             out_sems):
        my = lax.axis_index("i")
        left = lax.rem(my + (N_DEV - 1), N_DEV)
        right = lax.rem(my + 1, N_DEV)

        barrier_sem = pltpu.get_barrier_semaphore()
        for nbr in (left, right):
            pl.semaphore_signal(
                barrier_sem, inc=1, device_id=(nbr,),
                device_id_type=pl.DeviceIdType.MESH,
            )
        pl.semaphore_wait(barrier_sem, 2)

        def partial_chunk(c):
            xs = x_ref[pl.ds(c * MC, MC), :]
            return lax.dot_general(
                xs, w_ref[...], (((1,), (0,)), ((), ())),
                preferred_element_type=jnp.float32,
            )

        def rs_rdma(hop):
            return pltpu.make_async_remote_copy(
                src_ref=rs_send,
                dst_ref=rs_recv.at[hop],
                send_sem=rs_send_sem,
                recv_sem=rs_recv_sems.at[hop],
                device_id=(right,),
                device_id_type=pl.DeviceIdType.MESH,
            )

        p = partial_chunk(my)
        rs_send[...] = p.astype(jnp.bfloat16)
        rdma0 = rs_rdma(0)
        rdma0.start()
        p = partial_chunk(lax.rem(my + (N_DEV - 1), N_DEV))
        rdma0.wait_recv()
        rdma0.wait_send()
        rs_send[...] = (p + rs_recv[0].astype(jnp.float32)).astype(jnp.bfloat16)
        rdma1 = rs_rdma(1)
        rdma1.start()
        p = partial_chunk(lax.rem(my + (N_DEV - 2), N_DEV))
        rdma1.wait_recv()
        rdma1.wait_send()
        rs_send[...] = (p + rs_recv[1].astype(jnp.float32)).astype(jnp.bfloat16)
        rdma2 = rs_rdma(2)
        rdma2.start()
        p = partial_chunk(lax.rem(my + (N_DEV - 3), N_DEV))
        rdma2.wait_recv()
        c_own = lax.rem(my + 1, N_DEV)
        y_ref[pl.ds(c_own * MC, MC), :] = (
            p + rs_recv[2].astype(jnp.float32)
        ).astype(jnp.bfloat16)
        rdma2.wait_send()

        for h in range(N_DEV - 1):
            c_s = lax.rem(my + 1 + h, N_DEV)
            rows = pl.ds(c_s * MC, MC)
            ag = pltpu.make_async_remote_copy(
                src_ref=y_ref.at[rows, :],
                dst_ref=y_ref.at[rows, :],
                send_sem=ag_send_sems.at[h],
                recv_sem=ag_recv_sems.at[h],
                device_id=(left,),
                device_id_type=pl.DeviceIdType.MESH,
            )
            ag.start()
            ag.wait()

        amax = 1e-30
        for s in range(n_sub):
            amax = jnp.maximum(
                amax, jnp.max(y_ref[pl.ds(s * SC, SC), :].astype(jnp.float32))
            )
        scale = amax / 448.0
        inv = 448.0 / amax
        copies = []
        for s in range(n_sub):
            slot = s % 2
            if s >= 2:
                copies[s - 2].wait()
            r = jnp.maximum(y_ref[pl.ds(s * SC, SC), :], 0).astype(jnp.float32)
            q = jnp.minimum(r * inv, 448.0).astype(jnp.float8_e4m3fn)
            stage[slot, :, :] = q.astype(jnp.float32) * scale
            cp = pltpu.make_async_copy(
                stage.at[slot],
                out_ref.at[pl.ds(s * SC, SC), :],
                out_sems.at[slot],
            )
            cp.start()
            copies.append(cp)
        copies[n_sub - 2].wait()
        copies[n_sub - 1].wait()

    return pl.pallas_call(
        body,
        out_shape=jax.ShapeDtypeStruct((M, N), jnp.float32),
        in_specs=[
            pl.BlockSpec(memory_space=pltpu.VMEM),
            pl.BlockSpec(memory_space=pltpu.VMEM),
        ],
        out_specs=pl.BlockSpec(memory_space=pl.ANY),
        scratch_shapes=[
            pltpu.VMEM((M, N), jnp.bfloat16),
            pltpu.VMEM((N_DEV - 1, MC, N), jnp.bfloat16),
            pltpu.VMEM((MC, N), jnp.bfloat16),
            pltpu.VMEM((2, SC, N), jnp.float32),
            pltpu.SemaphoreType.DMA((N_DEV - 1,)),
            pltpu.SemaphoreType.DMA,
            pltpu.SemaphoreType.DMA((N_DEV - 1,)),
            pltpu.SemaphoreType.DMA((N_DEV - 1,)),
            pltpu.SemaphoreType.DMA((2,)),
        ],
        compiler_params=pltpu.CompilerParams(
            collective_id=0,
            vmem_limit_bytes=63 * 1024 * 1024,
        ),
    )(x, w)


# device time: 221257 ns/iter; 1.6278x vs baseline; 1.6278x over previous
import jax
import jax.numpy as jnp
from jax import lax
from jax.experimental import pallas as pl
from jax.experimental.pallas import tpu as pltpu

N_DEV = 4


def kernel(x, w_mat):
    M, _ = x.shape
    _, N = w_mat.shape
    MC = M // N_DEV
    NH = N // 2
    SC = 512
    n_sub = M // SC

    x = x.astype(jnp.bfloat16)
    w = w_mat.astype(jnp.bfloat16)

    def body(x_ref, w_ref, out_ref, y_ref,
             rsr_a, rsr_b, snd_a, snd_b, stage,
             rsr_a_sems, rsr_b_sems, snd_a_sem, snd_b_sem,
             ag_snd_a_sems, ag_rcv_a_sems, ag_snd_b_sems, ag_rcv_b_sems,
             out_sems):
        my = lax.axis_index("i")
        left = lax.rem(my + (N_DEV - 1), N_DEV)
        right = lax.rem(my + 1, N_DEV)

        def mod(c):
            return lax.rem(c + N_DEV, N_DEV)

        barrier_sem = pltpu.get_barrier_semaphore()
        for nbr in (left, right):
            pl.semaphore_signal(
                barrier_sem, inc=1, device_id=(nbr,),
                device_id_type=pl.DeviceIdType.MESH,
            )
        pl.semaphore_wait(barrier_sem, 2)

        def pchunk(c, cols):
            xs = x_ref[pl.ds(c * MC, MC), :]
            return lax.dot_general(
                xs, w_ref[:, cols], (((1,), (0,)), ((), ())),
                preferred_element_type=jnp.float32,
            )

        A = slice(0, NH)
        B = slice(NH, N)

        def rs_pair(hop):
            ra = pltpu.make_async_remote_copy(
                src_ref=snd_a, dst_ref=rsr_a.at[hop],
                send_sem=snd_a_sem, recv_sem=rsr_a_sems.at[hop],
                device_id=(right,), device_id_type=pl.DeviceIdType.MESH,
            )
            rb = pltpu.make_async_remote_copy(
                src_ref=snd_b, dst_ref=rsr_b.at[hop],
                send_sem=snd_b_sem, recv_sem=rsr_b_sems.at[hop],
                device_id=(left,), device_id_type=pl.DeviceIdType.MESH,
            )
            ra.start()
            rb.start()
            return ra, rb

        p = pchunk(my, slice(0, N))
        snd_a[...] = p[:, A].astype(jnp.bfloat16)
        snd_b[...] = p[:, B].astype(jnp.bfloat16)
        ra0, rb0 = rs_pair(0)
        pa = pchunk(mod(my - 1), A)
        pb = pchunk(mod(my + 1), B)
        ra0.wait()
        rb0.wait()
        snd_a[...] = (pa + rsr_a[0].astype(jnp.float32)).astype(jnp.bfloat16)
        snd_b[...] = (pb + rsr_b[0].astype(jnp.float32)).astype(jnp.bfloat16)
        ra1, rb1 = rs_pair(1)
        p = pchunk(mod(my + 2), slice(0, N))
        ra1.wait()
        rb1.wait()
        snd_a[...] = (p[:, A] + rsr_a[1].astype(jnp.float32)).astype(jnp.bfloat16)
        snd_b[...] = (p[:, B] + rsr_b[1].astype(jnp.float32)).astype(jnp.bfloat16)
        ra2, rb2 = rs_pair(2)
        pa = pchunk(mod(my + 1), A)
        pb = pchunk(mod(my - 1), B)
        ra2.wait()
        rb2.wait()
        own_a = mod(my + 1)
        own_b = mod(my - 1)
        y_ref[pl.ds(own_a * MC, MC), A] = (
            pa + rsr_a[2].astype(jnp.float32)
        ).astype(jnp.bfloat16)
        y_ref[pl.ds(own_b * MC, MC), B] = (
            pb + rsr_b[2].astype(jnp.float32)
        ).astype(jnp.bfloat16)

        for h in range(N_DEV - 1):
            c_a = mod(my + 1 + h)
            c_b = mod(my - 1 - h)
            ag_a = pltpu.make_async_remote_copy(
                src_ref=y_ref.at[pl.ds(c_a * MC, MC), A],
                dst_ref=y_ref.at[pl.ds(c_a * MC, MC), A],
                send_sem=ag_snd_a_sems.at[h], recv_sem=ag_rcv_a_sems.at[h],
                device_id=(left,), device_id_type=pl.DeviceIdType.MESH,
            )
            ag_b = pltpu.make_async_remote_copy(
                src_ref=y_ref.at[pl.ds(c_b * MC, MC), B],
                dst_ref=y_ref.at[pl.ds(c_b * MC, MC), B],
                send_sem=ag_snd_b_sems.at[h], recv_sem=ag_rcv_b_sems.at[h],
                device_id=(right,), device_id_type=pl.DeviceIdType.MESH,
            )
            ag_a.start()
            ag_b.start()
            ag_a.wait()
            ag_b.wait()

        amax = 1e-30
        for s in range(n_sub):
            amax = jnp.maximum(
                amax, jnp.max(y_ref[pl.ds(s * SC, SC), :].astype(jnp.float32))
            )
        scale = amax / 448.0
        inv = 448.0 / amax
        copies = []
        for s in range(n_sub):
            slot = s % 2
            if s >= 2:
                copies[s - 2].wait()
            r = jnp.maximum(y_ref[pl.ds(s * SC, SC), :], 0).astype(jnp.float32)
            q = jnp.minimum(r * inv, 448.0).astype(jnp.float8_e4m3fn)
            stage[slot, :, :] = q.astype(jnp.float32) * scale
            cp = pltpu.make_async_copy(
                stage.at[slot],
                out_ref.at[pl.ds(s * SC, SC), :],
                out_sems.at[slot],
            )
            cp.start()
            copies.append(cp)
        copies[n_sub - 2].wait()
        copies[n_sub - 1].wait()

    return pl.pallas_call(
        body,
        out_shape=jax.ShapeDtypeStruct((M, N), jnp.float32),
        in_specs=[
            pl.BlockSpec(memory_space=pltpu.VMEM),
            pl.BlockSpec(memory_space=pltpu.VMEM),
        ],
        out_specs=pl.BlockSpec(memory_space=pl.ANY),
        scratch_shapes=[
            pltpu.VMEM((M, N), jnp.bfloat16),
            pltpu.VMEM((N_DEV - 1, MC, NH), jnp.bfloat16),
            pltpu.VMEM((N_DEV - 1, MC, NH), jnp.bfloat16),
            pltpu.VMEM((MC, NH), jnp.bfloat16),
            pltpu.VMEM((MC, NH), jnp.bfloat16),
            pltpu.VMEM((2, SC, N), jnp.float32),
            pltpu.SemaphoreType.DMA((N_DEV - 1,)),
            pltpu.SemaphoreType.DMA((N_DEV - 1,)),
            pltpu.SemaphoreType.DMA,
            pltpu.SemaphoreType.DMA,
            pltpu.SemaphoreType.DMA((N_DEV - 1,)),
            pltpu.SemaphoreType.DMA((N_DEV - 1,)),
            pltpu.SemaphoreType.DMA((N_DEV - 1,)),
            pltpu.SemaphoreType.DMA((N_DEV - 1,)),
            pltpu.SemaphoreType.DMA((2,)),
        ],
        compiler_params=pltpu.CompilerParams(
            collective_id=0,
            vmem_limit_bytes=63 * 1024 * 1024,
        ),
    )(x, w)


# device time: 188152 ns/iter; 1.9142x vs baseline; 1.1759x over previous
import jax
import jax.numpy as jnp
from jax import lax
from jax.experimental import pallas as pl
from jax.experimental.pallas import tpu as pltpu

N_DEV = 4


def kernel(x, w_mat):
    M, _ = x.shape
    _, N = w_mat.shape
    MC = M // N_DEV
    NH = N // 2
    SC = 512
    n_sub = M // SC

    x = x.astype(jnp.bfloat16)
    w = w_mat.astype(jnp.bfloat16)

    def body(x_ref, w_ref, out_ref, q_ref,
             rsr_a, rsr_b, snd_a, snd_b, stage, amax_buf,
             rsr_a_sems, rsr_b_sems, snd_a_sem, snd_b_sem,
             ag_snd_a_sems, ag_rcv_a_sems, ag_snd_b_sems, ag_rcv_b_sems,
             amax_snd_sems, amax_rcv_sems, out_sems):
        my = lax.axis_index("i")
        left = lax.rem(my + (N_DEV - 1), N_DEV)
        right = lax.rem(my + 1, N_DEV)
        opp = lax.rem(my + 2, N_DEV)

        def mod(c):
            return lax.rem(c + N_DEV, N_DEV)

        barrier_sem = pltpu.get_barrier_semaphore()
        for nbr in (left, right):
            pl.semaphore_signal(
                barrier_sem, inc=1, device_id=(nbr,),
                device_id_type=pl.DeviceIdType.MESH,
            )
        pl.semaphore_wait(barrier_sem, 2)

        def pchunk(c, cols):
            xs = x_ref[pl.ds(c * MC, MC), :]
            return lax.dot_general(
                xs, w_ref[:, cols], (((1,), (0,)), ((), ())),
                preferred_element_type=jnp.float32,
            )

        A = slice(0, NH)
        B = slice(NH, N)

        def rs_pair(hop):
            ra = pltpu.make_async_remote_copy(
                src_ref=snd_a, dst_ref=rsr_a.at[hop],
                send_sem=snd_a_sem, recv_sem=rsr_a_sems.at[hop],
                device_id=(right,), device_id_type=pl.DeviceIdType.MESH,
            )
            rb = pltpu.make_async_remote_copy(
                src_ref=snd_b, dst_ref=rsr_b.at[hop],
                send_sem=snd_b_sem, recv_sem=rsr_b_sems.at[hop],
                device_id=(left,), device_id_type=pl.DeviceIdType.MESH,
            )
            ra.start()
            rb.start()
            return ra, rb

        p = pchunk(my, slice(0, N))
        snd_a[...] = p[:, A].astype(jnp.bfloat16)
        snd_b[...] = p[:, B].astype(jnp.bfloat16)
        ra0, rb0 = rs_pair(0)
        pa = pchunk(mod(my - 1), A)
        pb = pchunk(mod(my + 1), B)
        ra0.wait()
        rb0.wait()
        snd_a[...] = (pa + rsr_a[0].astype(jnp.float32)).astype(jnp.bfloat16)
        snd_b[...] = (pb + rsr_b[0].astype(jnp.float32)).astype(jnp.bfloat16)
        ra1, rb1 = rs_pair(1)
        p = pchunk(mod(my + 2), slice(0, N))
        ra1.wait()
        rb1.wait()
        snd_a[...] = (p[:, A] + rsr_a[1].astype(jnp.float32)).astype(jnp.bfloat16)
        snd_b[...] = (p[:, B] + rsr_b[1].astype(jnp.float32)).astype(jnp.bfloat16)
        ra2, rb2 = rs_pair(2)
        pa = pchunk(mod(my + 1), A)
        pb = pchunk(mod(my - 1), B)
        ra2.wait()
        rb2.wait()
        acc_a = pa + rsr_a[2].astype(jnp.float32)
        acc_b = pb + rsr_b[2].astype(jnp.float32)
        own_a = mod(my + 1)
        own_b = mod(my - 1)

        amax_local = jnp.maximum(jnp.max(acc_a), jnp.max(acc_b))
        amax_local = jnp.maximum(amax_local, 1e-30)
        amax_buf[pl.ds(my, 1), :, :] = jnp.full(
            (1, 8, 128), amax_local, jnp.float32
        )
        peers = (left, right, opp)
        excs = []
        for k, peer in enumerate(peers):
            exc = pltpu.make_async_remote_copy(
                src_ref=amax_buf.at[pl.ds(my, 1)],
                dst_ref=amax_buf.at[pl.ds(my, 1)],
                send_sem=amax_snd_sems.at[k], recv_sem=amax_rcv_sems.at[k],
                device_id=(peer,), device_id_type=pl.DeviceIdType.MESH,
            )
            exc.start()
            excs.append(exc)
        for exc in excs:
            exc.wait()
        amax = jnp.max(amax_buf[...])
        scale = amax / 448.0
        inv = 448.0 / amax

        def quant(acc):
            r = jnp.maximum(acc, 0.0)
            return jnp.minimum(r * inv, 448.0).astype(jnp.float8_e4m3fn)

        q_ref[pl.ds(own_a * MC, MC), A] = quant(acc_a)
        q_ref[pl.ds(own_b * MC, MC), B] = quant(acc_b)

        for h in range(N_DEV - 1):
            c_a = mod(my + 1 + h)
            c_b = mod(my - 1 - h)
            ag_a = pltpu.make_async_remote_copy(
                src_ref=q_ref.at[pl.ds(c_a * MC, MC), A],
                dst_ref=q_ref.at[pl.ds(c_a * MC, MC), A],
                send_sem=ag_snd_a_sems.at[h], recv_sem=ag_rcv_a_sems.at[h],
                device_id=(left,), device_id_type=pl.DeviceIdType.MESH,
            )
            ag_b = pltpu.make_async_remote_copy(
                src_ref=q_ref.at[pl.ds(c_b * MC, MC), B],
                dst_ref=q_ref.at[pl.ds(c_b * MC, MC), B],
                send_sem=ag_snd_b_sems.at[h], recv_sem=ag_rcv_b_sems.at[h],
                device_id=(right,), device_id_type=pl.DeviceIdType.MESH,
            )
            ag_a.start()
            ag_b.start()
            ag_a.wait()
            ag_b.wait()

        copies = []
        for s in range(n_sub):
            slot = s % 2
            if s >= 2:
                copies[s - 2].wait()
            qs = q_ref[pl.ds(s * SC, SC), :].astype(jnp.float32)
            stage[slot, :, :] = qs * scale
            cp = pltpu.make_async_copy(
                stage.at[slot],
                out_ref.at[pl.ds(s * SC, SC), :],
                out_sems.at[slot],
            )
            cp.start()
            copies.append(cp)
        copies[n_sub - 2].wait()
        copies[n_sub - 1].wait()

    return pl.pallas_call(
        body,
        out_shape=jax.ShapeDtypeStruct((M, N), jnp.float32),
        in_specs=[
            pl.BlockSpec(memory_space=pltpu.VMEM),
            pl.BlockSpec(memory_space=pltpu.VMEM),
        ],
        out_specs=pl.BlockSpec(memory_space=pl.ANY),
        scratch_shapes=[
            pltpu.VMEM((M, N), jnp.float8_e4m3fn),
            pltpu.VMEM((N_DEV - 1, MC, NH), jnp.bfloat16),
            pltpu.VMEM((N_DEV - 1, MC, NH), jnp.bfloat16),
            pltpu.VMEM((MC, NH), jnp.bfloat16),
            pltpu.VMEM((MC, NH), jnp.bfloat16),
            pltpu.VMEM((2, SC, N), jnp.float32),
            pltpu.VMEM((N_DEV, 8, 128), jnp.float32),
            pltpu.SemaphoreType.DMA((N_DEV - 1,)),
            pltpu.SemaphoreType.DMA((N_DEV - 1,)),
            pltpu.SemaphoreType.DMA,
            pltpu.SemaphoreType.DMA,
            pltpu.SemaphoreType.DMA((N_DEV - 1,)),
            pltpu.SemaphoreType.DMA((N_DEV - 1,)),
            pltpu.SemaphoreType.DMA((N_DEV - 1,)),
            pltpu.SemaphoreType.DMA((N_DEV - 1,)),
            pltpu.SemaphoreType.DMA((3,)),
            pltpu.SemaphoreType.DMA((3,)),
            pltpu.SemaphoreType.DMA((2,)),
        ],
        compiler_params=pltpu.CompilerParams(
            collective_id=0,
            vmem_limit_bytes=63 * 1024 * 1024,
        ),
    )(x, w)


# device time: 180282 ns/iter; 1.9978x vs baseline; 1.0437x over previous
import jax
import jax.numpy as jnp
from jax import lax
from jax.experimental import pallas as pl
from jax.experimental.pallas import tpu as pltpu

N_DEV = 4


def kernel(x, w_mat):
    M, _ = x.shape
    _, N = w_mat.shape
    MC = M // N_DEV
    NH = N // 2
    SC = 512
    n_sub = M // SC

    x = x.astype(jnp.bfloat16)
    w = w_mat.astype(jnp.bfloat16)

    def body(x_ref, w_ref, out_ref, q_ref,
             rsr_a, rsr_b, snd_a, snd_b, stage, amax_buf,
             rsr_a_sems, rsr_b_sems, snd_a_sem, snd_b_sem,
             ag_snd_a_sems, ag_rcv_a_sems, ag_snd_b_sems, ag_rcv_b_sems,
             amax_snd_sems, amax_rcv_sems, out_sems):
        my = lax.axis_index("i")
        left = lax.rem(my + (N_DEV - 1), N_DEV)
        right = lax.rem(my + 1, N_DEV)
        opp = lax.rem(my + 2, N_DEV)

        def mod(c):
            return lax.rem(c + N_DEV, N_DEV)

        barrier_sem = pltpu.get_barrier_semaphore()
        for nbr in (left, right):
            pl.semaphore_signal(
                barrier_sem, inc=1, device_id=(nbr,),
                device_id_type=pl.DeviceIdType.MESH,
            )
        pl.semaphore_wait(barrier_sem, 2)

        def pchunk(c, cols):
            xs = x_ref[pl.ds(c * MC, MC), :]
            return lax.dot_general(
                xs, w_ref[:, cols], (((1,), (0,)), ((), ())),
                preferred_element_type=jnp.float32,
            )

        A = slice(0, NH)
        B = slice(NH, N)

        def rs_pair(hop):
            ra = pltpu.make_async_remote_copy(
                src_ref=snd_a, dst_ref=rsr_a.at[hop],
                send_sem=snd_a_sem, recv_sem=rsr_a_sems.at[hop],
                device_id=(right,), device_id_type=pl.DeviceIdType.MESH,
            )
            rb = pltpu.make_async_remote_copy(
                src_ref=snd_b, dst_ref=rsr_b.at[hop],
                send_sem=snd_b_sem, recv_sem=rsr_b_sems.at[hop],
                device_id=(left,), device_id_type=pl.DeviceIdType.MESH,
            )
            ra.start()
            rb.start()
            return ra, rb

        p = pchunk(my, slice(0, N))
        snd_a[...] = p[:, A].astype(jnp.bfloat16)
        snd_b[...] = p[:, B].astype(jnp.bfloat16)
        ra0, rb0 = rs_pair(0)
        pa = pchunk(mod(my - 1), A)
        pb = pchunk(mod(my + 1), B)
        ra0.wait()
        rb0.wait()
        snd_a[...] = (pa + rsr_a[0].astype(jnp.float32)).astype(jnp.bfloat16)
        snd_b[...] = (pb + rsr_b[0].astype(jnp.float32)).astype(jnp.bfloat16)
        ra1, rb1 = rs_pair(1)
        p = pchunk(mod(my + 2), slice(0, N))
        ra1.wait()
        rb1.wait()
        snd_a[...] = (p[:, A] + rsr_a[1].astype(jnp.float32)).astype(jnp.bfloat16)
        snd_b[...] = (p[:, B] + rsr_b[1].astype(jnp.float32)).astype(jnp.bfloat16)
        ra2, rb2 = rs_pair(2)
        pa = pchunk(mod(my + 1), A)
        pb = pchunk(mod(my - 1), B)
        ra2.wait()
        rb2.wait()
        acc_a = pa + rsr_a[2].astype(jnp.float32)
        acc_b = pb + rsr_b[2].astype(jnp.float32)
        own_a = mod(my + 1)
        own_b = mod(my - 1)

        amax_local = jnp.maximum(jnp.max(acc_a), jnp.max(acc_b))
        amax_local = jnp.maximum(amax_local, 1e-30)
        amax_buf[pl.ds(my, 1), :, :] = jnp.full(
            (1, 8, 128), amax_local, jnp.float32
        )
        peers = (left, right, opp)
        excs = []
        for k, peer in enumerate(peers):
            exc = pltpu.make_async_remote_copy(
                src_ref=amax_buf.at[pl.ds(my, 1)],
                dst_ref=amax_buf.at[pl.ds(my, 1)],
                send_sem=amax_snd_sems.at[k], recv_sem=amax_rcv_sems.at[k],
                device_id=(peer,), device_id_type=pl.DeviceIdType.MESH,
            )
            exc.start()
            excs.append(exc)
        for exc in excs:
            exc.wait()
        amax = jnp.max(amax_buf[...])
        scale = amax / 448.0
        inv = 448.0 / amax

        def quant(acc):
            r = jnp.maximum(acc, 0.0)
            return jnp.minimum(r * inv, 448.0).astype(jnp.float8_e4m3fn)

        q_ref[pl.ds(own_a * MC, MC), A] = quant(acc_a)
        q_ref[pl.ds(own_b * MC, MC), B] = quant(acc_b)

        copies = []

        def emit(c, cols):
            slot = len(copies) % 2
            if len(copies) >= 2:
                copies[-2].wait()
            rows = pl.ds(c * MC, MC)
            stage[slot, :, :] = q_ref[rows, cols].astype(jnp.float32) * scale
            cp = pltpu.make_async_copy(
                stage.at[slot],
                out_ref.at[rows, cols],
                out_sems.at[slot],
            )
            cp.start()
            copies.append(cp)

        for h in range(N_DEV - 1):
            c_a = mod(my + 1 + h)
            c_b = mod(my - 1 - h)
            ag_a = pltpu.make_async_remote_copy(
                src_ref=q_ref.at[pl.ds(c_a * MC, MC), A],
                dst_ref=q_ref.at[pl.ds(c_a * MC, MC), A],
                send_sem=ag_snd_a_sems.at[h], recv_sem=ag_rcv_a_sems.at[h],
                device_id=(left,), device_id_type=pl.DeviceIdType.MESH,
            )
            ag_b = pltpu.make_async_remote_copy(
                src_ref=q_ref.at[pl.ds(c_b * MC, MC), B],
                dst_ref=q_ref.at[pl.ds(c_b * MC, MC), B],
                send_sem=ag_snd_b_sems.at[h], recv_sem=ag_rcv_b_sems.at[h],
                device_id=(right,), device_id_type=pl.DeviceIdType.MESH,
            )
            ag_a.start()
            ag_b.start()
            emit(c_a, A)
            emit(c_b, B)
            ag_a.wait()
            ag_b.wait()
        emit(my, A)
        emit(my, B)
        copies[-2].wait()
        copies[-1].wait()

    return pl.pallas_call(
        body,
        out_shape=jax.ShapeDtypeStruct((M, N), jnp.float32),
        in_specs=[
            pl.BlockSpec(memory_space=pltpu.VMEM),
            pl.BlockSpec(memory_space=pltpu.VMEM),
        ],
        out_specs=pl.BlockSpec(memory_space=pl.ANY),
        scratch_shapes=[
            pltpu.VMEM((M, N), jnp.float8_e4m3fn),
            pltpu.VMEM((N_DEV - 1, MC, NH), jnp.bfloat16),
            pltpu.VMEM((N_DEV - 1, MC, NH), jnp.bfloat16),
            pltpu.VMEM((MC, NH), jnp.bfloat16),
            pltpu.VMEM((MC, NH), jnp.bfloat16),
            pltpu.VMEM((2, MC, NH), jnp.float32),
            pltpu.VMEM((N_DEV, 8, 128), jnp.float32),
            pltpu.SemaphoreType.DMA((N_DEV - 1,)),
            pltpu.SemaphoreType.DMA((N_DEV - 1,)),
            pltpu.SemaphoreType.DMA,
            pltpu.SemaphoreType.DMA,
            pltpu.SemaphoreType.DMA((N_DEV - 1,)),
            pltpu.SemaphoreType.DMA((N_DEV - 1,)),
            pltpu.SemaphoreType.DMA((N_DEV - 1,)),
            pltpu.SemaphoreType.DMA((N_DEV - 1,)),
            pltpu.SemaphoreType.DMA((3,)),
            pltpu.SemaphoreType.DMA((3,)),
            pltpu.SemaphoreType.DMA((2,)),
        ],
        compiler_params=pltpu.CompilerParams(
            collective_id=0,
            vmem_limit_bytes=63 * 1024 * 1024,
        ),
    )(x, w)


# device time: 164634 ns/iter; 2.1877x vs baseline; 1.0950x over previous
import jax
import jax.numpy as jnp
from jax import lax
from jax.experimental import pallas as pl
from jax.experimental.pallas import tpu as pltpu

N_DEV = 4


def kernel(x, w_mat):
    M, _ = x.shape
    _, N = w_mat.shape
    MC = M // N_DEV
    NH = N // 2
    SC = 512
    n_sub = M // SC

    x = x.astype(jnp.bfloat16)
    w = w_mat.astype(jnp.bfloat16)

    def body(x_ref, w_ref, out_ref, q_ref,
             rsr_a, rsr_b, snd_a, snd_b, stage, amax_buf,
             rsr_a_sems, rsr_b_sems, snd_a_sems, snd_b_sems,
             ag_snd_a_sems, ag_rcv_a_sems, ag_snd_b_sems, ag_rcv_b_sems,
             amax_snd_sems, amax_rcv_sems, out_sems):
        my = lax.axis_index("i")
        left = lax.rem(my + (N_DEV - 1), N_DEV)
        right = lax.rem(my + 1, N_DEV)
        opp = lax.rem(my + 2, N_DEV)

        def mod(c):
            return lax.rem(c + N_DEV, N_DEV)

        barrier_sem = pltpu.get_barrier_semaphore()
        for nbr in (left, right):
            pl.semaphore_signal(
                barrier_sem, inc=1, device_id=(nbr,),
                device_id_type=pl.DeviceIdType.MESH,
            )
        pl.semaphore_wait(barrier_sem, 2)

        def pchunk(c, cols):
            xs = x_ref[pl.ds(c * MC, MC), :]
            return lax.dot_general(
                xs, w_ref[:, cols], (((1,), (0,)), ((), ())),
                preferred_element_type=jnp.float32,
            )

        A = slice(0, NH)
        B = slice(NH, N)

        SB = MC // 2

        def make_dir(snd, rsr, snd_sems, rsr_sems, dst):
            def mk(s, j):
                rows = pl.ds(j * SB, SB)
                return pltpu.make_async_remote_copy(
                    src_ref=snd.at[rows, :],
                    dst_ref=rsr.at[s, rows, :],
                    send_sem=snd_sems.at[j],
                    recv_sem=rsr_sems.at[s, j],
                    device_id=(dst,),
                    device_id_type=pl.DeviceIdType.MESH,
                )
            return mk

        mk_a = make_dir(snd_a, rsr_a, snd_a_sems, rsr_a_sems, right)
        mk_b = make_dir(snd_b, rsr_b, snd_b_sems, rsr_b_sems, left)
        rd_a = {}
        rd_b = {}

        for j in range(2):
            xs = x_ref[pl.ds(my * MC + j * SB, SB), :]
            pj = lax.dot_general(
                xs, w_ref[...], (((1,), (0,)), ((), ())),
                preferred_element_type=jnp.float32,
            )
            rows = pl.ds(j * SB, SB)
            snd_a[rows, :] = pj[:, A].astype(jnp.bfloat16)
            snd_b[rows, :] = pj[:, B].astype(jnp.bfloat16)
            rd_a[(0, j)] = mk_a(0, j)
            rd_b[(0, j)] = mk_b(0, j)
            rd_a[(0, j)].start()
            rd_b[(0, j)].start()

        pa = pchunk(mod(my - 1), A)
        pb = pchunk(mod(my + 1), B)
        for j in range(2):
            rows = pl.ds(j * SB, SB)
            lo, hi = j * SB, (j + 1) * SB
            rd_a[(0, j)].wait_recv()
            rd_a[(0, j)].wait_send()
            snd_a[rows, :] = (
                pa[lo:hi] + rsr_a[0, rows, :].astype(jnp.float32)
            ).astype(jnp.bfloat16)
            rd_a[(1, j)] = mk_a(1, j)
            rd_a[(1, j)].start()
            rd_b[(0, j)].wait_recv()
            rd_b[(0, j)].wait_send()
            snd_b[rows, :] = (
                pb[lo:hi] + rsr_b[0, rows, :].astype(jnp.float32)
            ).astype(jnp.bfloat16)
            rd_b[(1, j)] = mk_b(1, j)
            rd_b[(1, j)].start()

        p2 = pchunk(mod(my + 2), slice(0, N))
        for j in range(2):
            rows = pl.ds(j * SB, SB)
            lo, hi = j * SB, (j + 1) * SB
            rd_a[(1, j)].wait_recv()
            rd_a[(1, j)].wait_send()
            snd_a[rows, :] = (
                p2[lo:hi, A] + rsr_a[1, rows, :].astype(jnp.float32)
            ).astype(jnp.bfloat16)
            rd_a[(2, j)] = mk_a(2, j)
            rd_a[(2, j)].start()
            rd_b[(1, j)].wait_recv()
            rd_b[(1, j)].wait_send()
            snd_b[rows, :] = (
                p2[lo:hi, B] + rsr_b[1, rows, :].astype(jnp.float32)
            ).astype(jnp.bfloat16)
            rd_b[(2, j)] = mk_b(2, j)
            rd_b[(2, j)].start()

        pa_own = pchunk(mod(my + 1), A)
        pb_own = pchunk(mod(my - 1), B)
        acc_a = [None, None]
        acc_b = [None, None]
        amax_local = 1e-30
        for j in range(2):
            rows = pl.ds(j * SB, SB)
            lo, hi = j * SB, (j + 1) * SB
            rd_a[(2, j)].wait_recv()
            acc_a[j] = pa_own[lo:hi] + rsr_a[2, rows, :].astype(jnp.float32)
            amax_local = jnp.maximum(amax_local, jnp.max(acc_a[j]))
            rd_b[(2, j)].wait_recv()
            acc_b[j] = pb_own[lo:hi] + rsr_b[2, rows, :].astype(jnp.float32)
            amax_local = jnp.maximum(amax_local, jnp.max(acc_b[j]))
        for j in range(2):
            rd_a[(2, j)].wait_send()
            rd_b[(2, j)].wait_send()
        own_a = mod(my + 1)
        own_b = mod(my - 1)

        amax_buf[pl.ds(my, 1), :, :] = jnp.full(
            (1, 8, 128), amax_local, jnp.float32
        )
        peers = (left, right, opp)
        excs = []
        for k, peer in enumerate(peers):
            exc = pltpu.make_async_remote_copy(
                src_ref=amax_buf.at[pl.ds(my, 1)],
                dst_ref=amax_buf.at[pl.ds(my, 1)],
                send_sem=amax_snd_sems.at[k], recv_sem=amax_rcv_sems.at[k],
                device_id=(peer,), device_id_type=pl.DeviceIdType.MESH,
            )
            exc.start()
            excs.append(exc)
        for exc in excs:
            exc.wait()
        amax = jnp.max(amax_buf[...])
        scale = amax / 448.0
        inv = 448.0 / amax

        def quant(acc):
            r = jnp.maximum(acc, 0.0)
            return jnp.minimum(r * inv, 448.0).astype(jnp.float8_e4m3fn)

        for j in range(2):
            q_ref[pl.ds(own_a * MC + j * SB, SB), A] = quant(acc_a[j])
            q_ref[pl.ds(own_b * MC + j * SB, SB), B] = quant(acc_b[j])

        copies = []

        def emit(c, cols):
            slot = len(copies) % 2
            if len(copies) >= 2:
                copies[-2].wait()
            rows = pl.ds(c * MC, MC)
            stage[slot, :, :] = q_ref[rows, cols].astype(jnp.float32) * scale
            cp = pltpu.make_async_copy(
                stage.at[slot],
                out_ref.at[rows, cols],
                out_sems.at[slot],
            )
            cp.start()
            copies.append(cp)

        for h in range(N_DEV - 1):
            c_a = mod(my + 1 + h)
            c_b = mod(my - 1 - h)
            ag_a = pltpu.make_async_remote_copy(
                src_ref=q_ref.at[pl.ds(c_a * MC, MC), A],
                dst_ref=q_ref.at[pl.ds(c_a * MC, MC), A],
                send_sem=ag_snd_a_sems.at[h], recv_sem=ag_rcv_a_sems.at[h],
                device_id=(left,), device_id_type=pl.DeviceIdType.MESH,
            )
            ag_b = pltpu.make_async_remote_copy(
                src_ref=q_ref.at[pl.ds(c_b * MC, MC), B],
                dst_ref=q_ref.at[pl.ds(c_b * MC, MC), B],
                send_sem=ag_snd_b_sems.at[h], recv_sem=ag_rcv_b_sems.at[h],
                device_id=(right,), device_id_type=pl.DeviceIdType.MESH,
            )
            ag_a.start()
            ag_b.start()
            emit(c_a, A)
            emit(c_b, B)
            ag_a.wait()
            ag_b.wait()
        emit(my, A)
        emit(my, B)
        copies[-2].wait()
        copies[-1].wait()

    return pl.pallas_call(
        body,
        out_shape=jax.ShapeDtypeStruct((M, N), jnp.float32),
        in_specs=[
            pl.BlockSpec(memory_space=pltpu.VMEM),
            pl.BlockSpec(memory_space=pltpu.VMEM),
        ],
        out_specs=pl.BlockSpec(memory_space=pl.ANY),
        scratch_shapes=[
            pltpu.VMEM((M, N), jnp.float8_e4m3fn),
            pltpu.VMEM((N_DEV - 1, MC, NH), jnp.bfloat16),
            pltpu.VMEM((N_DEV - 1, MC, NH), jnp.bfloat16),
            pltpu.VMEM((MC, NH), jnp.bfloat16),
            pltpu.VMEM((MC, NH), jnp.bfloat16),
            pltpu.VMEM((2, MC, NH), jnp.float32),
            pltpu.VMEM((N_DEV, 8, 128), jnp.float32),
            pltpu.SemaphoreType.DMA((N_DEV - 1, 2)),
            pltpu.SemaphoreType.DMA((N_DEV - 1, 2)),
            pltpu.SemaphoreType.DMA((2,)),
            pltpu.SemaphoreType.DMA((2,)),
            pltpu.SemaphoreType.DMA((N_DEV - 1,)),
            pltpu.SemaphoreType.DMA((N_DEV - 1,)),
            pltpu.SemaphoreType.DMA((N_DEV - 1,)),
            pltpu.SemaphoreType.DMA((N_DEV - 1,)),
            pltpu.SemaphoreType.DMA((3,)),
            pltpu.SemaphoreType.DMA((3,)),
            pltpu.SemaphoreType.DMA((2,)),
        ],
        compiler_params=pltpu.CompilerParams(
            collective_id=0,
            vmem_limit_bytes=63 * 1024 * 1024,
        ),
    )(x, w)


# device time: 154066 ns/iter; 2.3377x vs baseline; 1.0686x over previous
import jax
import jax.numpy as jnp
from jax import lax
from jax.experimental import pallas as pl
from jax.experimental.pallas import tpu as pltpu

N_DEV = 4


def kernel(x, w_mat):
    M, _ = x.shape
    _, N = w_mat.shape
    MC = M // N_DEV
    NH = N // 2
    SC = 512
    n_sub = M // SC

    x = x.astype(jnp.bfloat16)
    w = w_mat.astype(jnp.bfloat16)

    def body(x_ref, w_ref, out_ref, q_ref,
             rsr_a, rsr_b, snd_a, snd_b, stage, amax_buf,
             rsr_a_sems, rsr_b_sems, snd_a_sems, snd_b_sems,
             ag_snd_a_sems, ag_rcv_a_sems, ag_snd_b_sems, ag_rcv_b_sems,
             amax_snd_sems, amax_rcv_sems, out_sems):
        my = lax.axis_index("i")
        left = lax.rem(my + (N_DEV - 1), N_DEV)
        right = lax.rem(my + 1, N_DEV)
        opp = lax.rem(my + 2, N_DEV)

        def mod(c):
            return lax.rem(c + N_DEV, N_DEV)

        barrier_sem = pltpu.get_barrier_semaphore()
        for nbr in (left, right):
            pl.semaphore_signal(
                barrier_sem, inc=1, device_id=(nbr,),
                device_id_type=pl.DeviceIdType.MESH,
            )
        pl.semaphore_wait(barrier_sem, 2)

        def pchunk(c, cols):
            xs = x_ref[pl.ds(c * MC, MC), :]
            return lax.dot_general(
                xs, w_ref[:, cols], (((1,), (0,)), ((), ())),
                preferred_element_type=jnp.float32,
            )

        A = slice(0, NH)
        B = slice(NH, N)

        SB = MC // 2

        def make_dir(snd, rsr, snd_sems, rsr_sems, dst):
            def mk(s, j):
                rows = pl.ds(j * SB, SB)
                return pltpu.make_async_remote_copy(
                    src_ref=snd.at[rows, :],
                    dst_ref=rsr.at[s, rows, :],
                    send_sem=snd_sems.at[j],
                    recv_sem=rsr_sems.at[s, j],
                    device_id=(dst,),
                    device_id_type=pl.DeviceIdType.MESH,
                )
            return mk

        mk_a = make_dir(snd_a, rsr_a, snd_a_sems, rsr_a_sems, right)
        mk_b = make_dir(snd_b, rsr_b, snd_b_sems, rsr_b_sems, left)
        rd_a = {}
        rd_b = {}

        for j in range(2):
            xs = x_ref[pl.ds(my * MC + j * SB, SB), :]
            pj = lax.dot_general(
                xs, w_ref[...], (((1,), (0,)), ((), ())),
                preferred_element_type=jnp.float32,
            )
            rows = pl.ds(j * SB, SB)
            snd_a[rows, :] = pj[:, A].astype(jnp.bfloat16)
            snd_b[rows, :] = pj[:, B].astype(jnp.bfloat16)
            rd_a[(0, j)] = mk_a(0, j)
            rd_b[(0, j)] = mk_b(0, j)
            rd_a[(0, j)].start()
            rd_b[(0, j)].start()

        pa = pchunk(mod(my - 1), A)
        pb = pchunk(mod(my + 1), B)
        for j in range(2):
            rows = pl.ds(j * SB, SB)
            lo, hi = j * SB, (j + 1) * SB
            rd_a[(0, j)].wait_recv()
            rd_a[(0, j)].wait_send()
            snd_a[rows, :] = (
                pa[lo:hi] + rsr_a[0, rows, :].astype(jnp.float32)
            ).astype(jnp.bfloat16)
            rd_a[(1, j)] = mk_a(1, j)
            rd_a[(1, j)].start()
            rd_b[(0, j)].wait_recv()
            rd_b[(0, j)].wait_send()
            snd_b[rows, :] = (
                pb[lo:hi] + rsr_b[0, rows, :].astype(jnp.float32)
            ).astype(jnp.bfloat16)
            rd_b[(1, j)] = mk_b(1, j)
            rd_b[(1, j)].start()

        p2 = pchunk(mod(my + 2), slice(0, N))
        for j in range(2):
            rows = pl.ds(j * SB, SB)
            lo, hi = j * SB, (j + 1) * SB
            rd_a[(1, j)].wait_recv()
            rd_a[(1, j)].wait_send()
            snd_a[rows, :] = (
                p2[lo:hi, A] + rsr_a[1, rows, :].astype(jnp.float32)
            ).astype(jnp.bfloat16)
            rd_a[(2, j)] = mk_a(2, j)
            rd_a[(2, j)].start()
            rd_b[(1, j)].wait_recv()
            rd_b[(1, j)].wait_send()
            snd_b[rows, :] = (
                p2[lo:hi, B] + rsr_b[1, rows, :].astype(jnp.float32)
            ).astype(jnp.bfloat16)
            rd_b[(2, j)] = mk_b(2, j)
            rd_b[(2, j)].start()

        pa_own = pchunk(mod(my + 1), A)
        pb_own = pchunk(mod(my - 1), B)
        acc_a = [None, None]
        acc_b = [None, None]
        amax_local = 1e-30
        for j in range(2):
            rows = pl.ds(j * SB, SB)
            lo, hi = j * SB, (j + 1) * SB
            rd_a[(2, j)].wait_recv()
            acc_a[j] = pa_own[lo:hi] + rsr_a[2, rows, :].astype(jnp.float32)
            amax_local = jnp.maximum(amax_local, jnp.max(acc_a[j]))
            rd_b[(2, j)].wait_recv()
            acc_b[j] = pb_own[lo:hi] + rsr_b[2, rows, :].astype(jnp.float32)
            amax_local = jnp.maximum(amax_local, jnp.max(acc_b[j]))
        for j in range(2):
            rd_a[(2, j)].wait_send()
            rd_b[(2, j)].wait_send()
        own_a = mod(my + 1)
        own_b = mod(my - 1)

        amax_buf[pl.ds(my, 1), :, :] = jnp.full(
            (1, 8, 128), amax_local, jnp.float32
        )
        peers = (left, right, opp)
        excs = []
        for k, peer in enumerate(peers):
            exc = pltpu.make_async_remote_copy(
                src_ref=amax_buf.at[pl.ds(my, 1)],
                dst_ref=amax_buf.at[pl.ds(my, 1)],
                send_sem=amax_snd_sems.at[k], recv_sem=amax_rcv_sems.at[k],
                device_id=(peer,), device_id_type=pl.DeviceIdType.MESH,
            )
            exc.start()
            excs.append(exc)
        for exc in excs:
            exc.wait()
        amax = jnp.max(amax_buf[...])
        scale = amax / 448.0
        inv = 448.0 / amax

        def quant(acc):
            r = jnp.maximum(acc, 0.0)
            return jnp.minimum(r * inv, 448.0).astype(jnp.float8_e4m3fn)

        for j in range(2):
            q_ref[pl.ds(own_a * MC + j * SB, SB), A] = quant(acc_a[j])
            q_ref[pl.ds(own_b * MC + j * SB, SB), B] = quant(acc_b[j])

        copies = []

        def emit(c, cols):
            slot = len(copies) % 2
            if len(copies) >= 2:
                copies[-2].wait()
            rows = pl.ds(c * MC, MC)
            stage[slot, :, :] = (
                q_ref[rows, cols].astype(jnp.float32) * scale
            ).astype(jnp.bfloat16)
            cp = pltpu.make_async_copy(
                stage.at[slot],
                out_ref.at[rows, cols],
                out_sems.at[slot],
            )
            cp.start()
            copies.append(cp)

        for h in range(N_DEV - 1):
            c_a = mod(my + 1 + h)
            c_b = mod(my - 1 - h)
            ag_a = pltpu.make_async_remote_copy(
                src_ref=q_ref.at[pl.ds(c_a * MC, MC), A],
                dst_ref=q_ref.at[pl.ds(c_a * MC, MC), A],
                send_sem=ag_snd_a_sems.at[h], recv_sem=ag_rcv_a_sems.at[h],
                device_id=(left,), device_id_type=pl.DeviceIdType.MESH,
            )
            ag_b = pltpu.make_async_remote_copy(
                src_ref=q_ref.at[pl.ds(c_b * MC, MC), B],
                dst_ref=q_ref.at[pl.ds(c_b * MC, MC), B],
                send_sem=ag_snd_b_sems.at[h], recv_sem=ag_rcv_b_sems.at[h],
                device_id=(right,), device_id_type=pl.DeviceIdType.MESH,
            )
            ag_a.start()
            ag_b.start()
            emit(c_a, A)
            emit(c_b, B)
            ag_a.wait()
            ag_b.wait()
        emit(my, A)
        emit(my, B)
        copies[-2].wait()
        copies[-1].wait()

    return pl.pallas_call(
        body,
        out_shape=jax.ShapeDtypeStruct((M, N), jnp.bfloat16),
        in_specs=[
            pl.BlockSpec(memory_space=pltpu.VMEM),
            pl.BlockSpec(memory_space=pltpu.VMEM),
        ],
        out_specs=pl.BlockSpec(memory_space=pl.ANY),
        scratch_shapes=[
            pltpu.VMEM((M, N), jnp.float8_e4m3fn),
            pltpu.VMEM((N_DEV - 1, MC, NH), jnp.bfloat16),
            pltpu.VMEM((N_DEV - 1, MC, NH), jnp.bfloat16),
            pltpu.VMEM((MC, NH), jnp.bfloat16),
            pltpu.VMEM((MC, NH), jnp.bfloat16),
            pltpu.VMEM((2, MC, NH), jnp.bfloat16),
            pltpu.VMEM((N_DEV, 8, 128), jnp.float32),
            pltpu.SemaphoreType.DMA((N_DEV - 1, 2)),
            pltpu.SemaphoreType.DMA((N_DEV - 1, 2)),
            pltpu.SemaphoreType.DMA((2,)),
            pltpu.SemaphoreType.DMA((2,)),
            pltpu.SemaphoreType.DMA((N_DEV - 1,)),
            pltpu.SemaphoreType.DMA((N_DEV - 1,)),
            pltpu.SemaphoreType.DMA((N_DEV - 1,)),
            pltpu.SemaphoreType.DMA((N_DEV - 1,)),
            pltpu.SemaphoreType.DMA((3,)),
            pltpu.SemaphoreType.DMA((3,)),
            pltpu.SemaphoreType.DMA((2,)),
        ],
        compiler_params=pltpu.CompilerParams(
            collective_id=0,
            vmem_limit_bytes=63 * 1024 * 1024,
        ),
    )(x, w)


# device time: 143118 ns/iter; 2.5166x vs baseline; 1.0765x over previous
import jax
import jax.numpy as jnp
from jax import lax
from jax.experimental import pallas as pl
from jax.experimental.pallas import tpu as pltpu

N_DEV = 4


def kernel(x, w_mat):
    M, _ = x.shape
    _, N = w_mat.shape
    MC = M // N_DEV
    NH = N // 2
    SC = 512
    n_sub = M // SC

    w = w_mat

    def body(x_ref, w_ref, out_ref, q_ref,
             rsr_a, rsr_b, snd_a, snd_b, stage, amax_buf,
             rsr_a_sems, rsr_b_sems, snd_a_sems, snd_b_sems,
             ag_snd_a_sems, ag_rcv_a_sems, ag_snd_b_sems, ag_rcv_b_sems,
             amax_snd_sems, amax_rcv_sems, out_sems):
        my = lax.axis_index("i")
        left = lax.rem(my + (N_DEV - 1), N_DEV)
        right = lax.rem(my + 1, N_DEV)
        opp = lax.rem(my + 2, N_DEV)

        def mod(c):
            return lax.rem(c + N_DEV, N_DEV)

        barrier_sem = pltpu.get_barrier_semaphore()
        for nbr in (left, right):
            pl.semaphore_signal(
                barrier_sem, inc=1, device_id=(nbr,),
                device_id_type=pl.DeviceIdType.MESH,
            )
        pl.semaphore_wait(barrier_sem, 2)

        def pchunk(c, cols):
            xs = x_ref[pl.ds(c * MC, MC), :].astype(jnp.bfloat16)
            ws = w_ref[:, cols].astype(jnp.bfloat16)
            return lax.dot_general(
                xs, ws, (((1,), (0,)), ((), ())),
                preferred_element_type=jnp.float32,
            )

        A = slice(0, NH)
        B = slice(NH, N)

        SB = MC // 2

        def make_dir(snd, rsr, snd_sems, rsr_sems, dst):
            def mk(s, j):
                rows = pl.ds(j * SB, SB)
                return pltpu.make_async_remote_copy(
                    src_ref=snd.at[rows, :],
                    dst_ref=rsr.at[s, rows, :],
                    send_sem=snd_sems.at[j],
                    recv_sem=rsr_sems.at[s, j],
                    device_id=(dst,),
                    device_id_type=pl.DeviceIdType.MESH,
                )
            return mk

        mk_a = make_dir(snd_a, rsr_a, snd_a_sems, rsr_a_sems, right)
        mk_b = make_dir(snd_b, rsr_b, snd_b_sems, rsr_b_sems, left)
        rd_a = {}
        rd_b = {}

        for j in range(2):
            xs = x_ref[pl.ds(my * MC + j * SB, SB), :].astype(jnp.bfloat16)
            pj = lax.dot_general(
                xs, w_ref[...].astype(jnp.bfloat16),
                (((1,), (0,)), ((), ())),
                preferred_element_type=jnp.float32,
            )
            rows = pl.ds(j * SB, SB)
            snd_a[rows, :] = pj[:, A].astype(jnp.bfloat16)
            snd_b[rows, :] = pj[:, B].astype(jnp.bfloat16)
            rd_a[(0, j)] = mk_a(0, j)
            rd_b[(0, j)] = mk_b(0, j)
            rd_a[(0, j)].start()
            rd_b[(0, j)].start()

        pa = pchunk(mod(my - 1), A)
        pb = pchunk(mod(my + 1), B)
        for j in range(2):
            rows = pl.ds(j * SB, SB)
            lo, hi = j * SB, (j + 1) * SB
            rd_a[(0, j)].wait_recv()
            rd_a[(0, j)].wait_send()
            snd_a[rows, :] = (
                pa[lo:hi] + rsr_a[0, rows, :].astype(jnp.float32)
            ).astype(jnp.bfloat16)
            rd_a[(1, j)] = mk_a(1, j)
            rd_a[(1, j)].start()
            rd_b[(0, j)].wait_recv()
            rd_b[(0, j)].wait_send()
            snd_b[rows, :] = (
                pb[lo:hi] + rsr_b[0, rows, :].astype(jnp.float32)
            ).astype(jnp.bfloat16)
            rd_b[(1, j)] = mk_b(1, j)
            rd_b[(1, j)].start()

        p2 = pchunk(mod(my + 2), slice(0, N))
        for j in range(2):
            rows = pl.ds(j * SB, SB)
            lo, hi = j * SB, (j + 1) * SB
            rd_a[(1, j)].wait_recv()
            rd_a[(1, j)].wait_send()
            snd_a[rows, :] = (
                p2[lo:hi, A] + rsr_a[1, rows, :].astype(jnp.float32)
            ).astype(jnp.bfloat16)
            rd_a[(2, j)] = mk_a(2, j)
            rd_a[(2, j)].start()
            rd_b[(1, j)].wait_recv()
            rd_b[(1, j)].wait_send()
            snd_b[rows, :] = (
                p2[lo:hi, B] + rsr_b[1, rows, :].astype(jnp.float32)
            ).astype(jnp.bfloat16)
            rd_b[(2, j)] = mk_b(2, j)
            rd_b[(2, j)].start()

        pa_own = pchunk(mod(my + 1), A)
        pb_own = pchunk(mod(my - 1), B)
        acc_a = [None, None]
        acc_b = [None, None]
        amax_local = 1e-30
        for j in range(2):
            rows = pl.ds(j * SB, SB)
            lo, hi = j * SB, (j + 1) * SB
            rd_a[(2, j)].wait_recv()
            acc_a[j] = pa_own[lo:hi] + rsr_a[2, rows, :].astype(jnp.float32)
            amax_local = jnp.maximum(amax_local, jnp.max(acc_a[j]))
            rd_b[(2, j)].wait_recv()
            acc_b[j] = pb_own[lo:hi] + rsr_b[2, rows, :].astype(jnp.float32)
            amax_local = jnp.maximum(amax_local, jnp.max(acc_b[j]))
        for j in range(2):
            rd_a[(2, j)].wait_send()
            rd_b[(2, j)].wait_send()
        own_a = mod(my + 1)
        own_b = mod(my - 1)

        amax_buf[pl.ds(my, 1), :, :] = jnp.full(
            (1, 8, 128), amax_local, jnp.float32
        )
        peers = (left, right, opp)
        excs = []
        for k, peer in enumerate(peers):
            exc = pltpu.make_async_remote_copy(
                src_ref=amax_buf.at[pl.ds(my, 1)],
                dst_ref=amax_buf.at[pl.ds(my, 1)],
                send_sem=amax_snd_sems.at[k], recv_sem=amax_rcv_sems.at[k],
                device_id=(peer,), device_id_type=pl.DeviceIdType.MESH,
            )
            exc.start()
            excs.append(exc)
        for exc in excs:
            exc.wait()
        amax = jnp.max(amax_buf[...])
        scale = amax / 448.0
        inv = 448.0 / amax

        def quant(acc):
            r = jnp.maximum(acc, 0.0)
            return jnp.minimum(r * inv, 448.0).astype(jnp.float8_e4m3fn)

        for j in range(2):
            q_ref[pl.ds(own_a * MC + j * SB, SB), A] = quant(acc_a[j])
            q_ref[pl.ds(own_b * MC + j * SB, SB), B] = quant(acc_b[j])

        copies = []

        def emit(c, cols):
            slot = len(copies) % 2
            if len(copies) >= 2:
                copies[-2].wait()
            rows = pl.ds(c * MC, MC)
            stage[slot, :, :] = (
                q_ref[rows, cols].astype(jnp.float32) * scale
            ).astype(jnp.bfloat16)
            cp = pltpu.make_async_copy(
                stage.at[slot],
                out_ref.at[rows, cols],
                out_sems.at[slot],
            )
            cp.start()
            copies.append(cp)

        for h in range(N_DEV - 1):
            c_a = mod(my + 1 + h)
            c_b = mod(my - 1 - h)
            ag_a = pltpu.make_async_remote_copy(
                src_ref=q_ref.at[pl.ds(c_a * MC, MC), A],
                dst_ref=q_ref.at[pl.ds(c_a * MC, MC), A],
                send_sem=ag_snd_a_sems.at[h], recv_sem=ag_rcv_a_sems.at[h],
                device_id=(left,), device_id_type=pl.DeviceIdType.MESH,
            )
            ag_b = pltpu.make_async_remote_copy(
                src_ref=q_ref.at[pl.ds(c_b * MC, MC), B],
                dst_ref=q_ref.at[pl.ds(c_b * MC, MC), B],
                send_sem=ag_snd_b_sems.at[h], recv_sem=ag_rcv_b_sems.at[h],
                device_id=(right,), device_id_type=pl.DeviceIdType.MESH,
            )
            ag_a.start()
            ag_b.start()
            emit(c_a, A)
            emit(c_b, B)
            ag_a.wait()
            ag_b.wait()
        emit(my, A)
        emit(my, B)
        copies[-2].wait()
        copies[-1].wait()

    return pl.pallas_call(
        body,
        out_shape=jax.ShapeDtypeStruct((M, N), jnp.bfloat16),
        in_specs=[
            pl.BlockSpec(memory_space=pltpu.VMEM),
            pl.BlockSpec(memory_space=pltpu.VMEM),
        ],
        out_specs=pl.BlockSpec(memory_space=pl.ANY),
        scratch_shapes=[
            pltpu.VMEM((M, N), jnp.float8_e4m3fn),
            pltpu.VMEM((N_DEV - 1, MC, NH), jnp.bfloat16),
            pltpu.VMEM((N_DEV - 1, MC, NH), jnp.bfloat16),
            pltpu.VMEM((MC, NH), jnp.bfloat16),
            pltpu.VMEM((MC, NH), jnp.bfloat16),
            pltpu.VMEM((2, MC, NH), jnp.bfloat16),
            pltpu.VMEM((N_DEV, 8, 128), jnp.float32),
            pltpu.SemaphoreType.DMA((N_DEV - 1, 2)),
            pltpu.SemaphoreType.DMA((N_DEV - 1, 2)),
            pltpu.SemaphoreType.DMA((2,)),
            pltpu.SemaphoreType.DMA((2,)),
            pltpu.SemaphoreType.DMA((N_DEV - 1,)),
            pltpu.SemaphoreType.DMA((N_DEV - 1,)),
            pltpu.SemaphoreType.DMA((N_DEV - 1,)),
            pltpu.SemaphoreType.DMA((N_DEV - 1,)),
            pltpu.SemaphoreType.DMA((3,)),
            pltpu.SemaphoreType.DMA((3,)),
            pltpu.SemaphoreType.DMA((2,)),
        ],
        compiler_params=pltpu.CompilerParams(
            collective_id=0,
            vmem_limit_bytes=63 * 1024 * 1024,
        ),
    )(x, w)


# device time: 139754 ns/iter; 2.5771x vs baseline; 1.0241x over previous
import jax
import jax.numpy as jnp
from jax import lax
from jax.experimental import pallas as pl
from jax.experimental.pallas import tpu as pltpu

N_DEV = 4


def kernel(x, w_mat):
    M, _ = x.shape
    _, N = w_mat.shape
    MC = M // N_DEV
    NH = N // 2
    SC = 512
    n_sub = M // SC

    w = w_mat

    def body(x_ref, w_ref, out_ref, q_ref,
             rsr_a, rsr_b, snd_a, snd_b, stage, amax_buf,
             rsr_a_sems, rsr_b_sems, snd_a_sems, snd_b_sems,
             ag_snd_a_sems, ag_rcv_a_sems, ag_snd_b_sems, ag_rcv_b_sems,
             amax_snd_sems, amax_rcv_sems, out_sems):
        my = lax.axis_index("i")
        left = lax.rem(my + (N_DEV - 1), N_DEV)
        right = lax.rem(my + 1, N_DEV)
        opp = lax.rem(my + 2, N_DEV)

        def mod(c):
            return lax.rem(c + N_DEV, N_DEV)

        barrier_sem = pltpu.get_barrier_semaphore()
        for nbr in (left, right):
            pl.semaphore_signal(
                barrier_sem, inc=1, device_id=(nbr,),
                device_id_type=pl.DeviceIdType.MESH,
            )
        pl.semaphore_wait(barrier_sem, 2)

        def pchunk(c, cols):
            xs = x_ref[pl.ds(c * MC, MC), :].astype(jnp.bfloat16)
            ws = w_ref[:, cols].astype(jnp.bfloat16)
            return lax.dot_general(
                xs, ws, (((1,), (0,)), ((), ())),
                preferred_element_type=jnp.float32,
            )

        A = slice(0, NH)
        B = slice(NH, N)

        SB = MC // 2

        def make_dir(snd, rsr, snd_sems, rsr_sems, dst):
            def mk(s, j):
                rows = pl.ds(j * SB, SB)
                return pltpu.make_async_remote_copy(
                    src_ref=snd.at[rows, :],
                    dst_ref=rsr.at[s, rows, :],
                    send_sem=snd_sems.at[j],
                    recv_sem=rsr_sems.at[s, j],
                    device_id=(dst,),
                    device_id_type=pl.DeviceIdType.MESH,
                )
            return mk

        mk_a = make_dir(snd_a, rsr_a, snd_a_sems, rsr_a_sems, right)
        mk_b = make_dir(snd_b, rsr_b, snd_b_sems, rsr_b_sems, left)
        rd_a = {}
        rd_b = {}

        for j in range(2):
            xs = x_ref[pl.ds(my * MC + j * SB, SB), :].astype(jnp.bfloat16)
            pj = lax.dot_general(
                xs, w_ref[...].astype(jnp.bfloat16),
                (((1,), (0,)), ((), ())),
                preferred_element_type=jnp.float32,
            )
            rows = pl.ds(j * SB, SB)
            snd_a[rows, :] = pj[:, A].astype(jnp.bfloat16)
            snd_b[rows, :] = pj[:, B].astype(jnp.bfloat16)
            rd_a[(0, j)] = mk_a(0, j)
            rd_b[(0, j)] = mk_b(0, j)
            rd_a[(0, j)].start()
            rd_b[(0, j)].start()

        pa = pchunk(mod(my - 1), A)
        pb = pchunk(mod(my + 1), B)
        for j in range(2):
            rows = pl.ds(j * SB, SB)
            lo, hi = j * SB, (j + 1) * SB
            rd_a[(0, j)].wait_recv()
            rd_a[(0, j)].wait_send()
            snd_a[rows, :] = (
                pa[lo:hi] + rsr_a[0, rows, :].astype(jnp.float32)
            ).astype(jnp.bfloat16)
            rd_a[(1, j)] = mk_a(1, j)
            rd_a[(1, j)].start()
            rd_b[(0, j)].wait_recv()
            rd_b[(0, j)].wait_send()
            snd_b[rows, :] = (
                pb[lo:hi] + rsr_b[0, rows, :].astype(jnp.float32)
            ).astype(jnp.bfloat16)
            rd_b[(1, j)] = mk_b(1, j)
            rd_b[(1, j)].start()

        p2 = pchunk(mod(my + 2), slice(0, N))
        for j in range(2):
            rows = pl.ds(j * SB, SB)
            lo, hi = j * SB, (j + 1) * SB
            rd_a[(1, j)].wait_recv()
            rd_a[(1, j)].wait_send()
            snd_a[rows, :] = (
                p2[lo:hi, A] + rsr_a[1, rows, :].astype(jnp.float32)
            ).astype(jnp.bfloat16)
            rd_a[(2, j)] = mk_a(2, j)
            rd_a[(2, j)].start()
            rd_b[(1, j)].wait_recv()
            rd_b[(1, j)].wait_send()
            snd_b[rows, :] = (
                p2[lo:hi, B] + rsr_b[1, rows, :].astype(jnp.float32)
            ).astype(jnp.bfloat16)
            rd_b[(2, j)] = mk_b(2, j)
            rd_b[(2, j)].start()

        pa_own = pchunk(mod(my + 1), A)
        pb_own = pchunk(mod(my - 1), B)
        acc_a = [None, None]
        acc_b = [None, None]
        amax_local = 1e-30
        for j in range(2):
            rows = pl.ds(j * SB, SB)
            lo, hi = j * SB, (j + 1) * SB
            rd_a[(2, j)].wait_recv()
            acc_a[j] = pa_own[lo:hi] + rsr_a[2, rows, :].astype(jnp.float32)
            amax_local = jnp.maximum(amax_local, jnp.max(acc_a[j]))
            rd_b[(2, j)].wait_recv()
            acc_b[j] = pb_own[lo:hi] + rsr_b[2, rows, :].astype(jnp.float32)
            amax_local = jnp.maximum(amax_local, jnp.max(acc_b[j]))
        for j in range(2):
            rd_a[(2, j)].wait_send()
            rd_b[(2, j)].wait_send()
        own_a = mod(my + 1)
        own_b = mod(my - 1)

        amax_buf[pl.ds(my, 1), :, :] = jnp.full(
            (1, 8, 128), amax_local, jnp.float32
        )
        peers = (left, right, opp)
        excs = []
        for k, peer in enumerate(peers):
            exc = pltpu.make_async_remote_copy(
                src_ref=amax_buf.at[pl.ds(my, 1)],
                dst_ref=amax_buf.at[pl.ds(my, 1)],
                send_sem=amax_snd_sems.at[k], recv_sem=amax_rcv_sems.at[k],
                device_id=(peer,), device_id_type=pl.DeviceIdType.MESH,
            )
            exc.start()
            excs.append(exc)
        for exc in excs:
            exc.wait()
        amax = jnp.max(amax_buf[...])
        scale = amax / 448.0
        inv = 448.0 / amax

        def quant(acc):
            r = jnp.maximum(acc, 0.0)
            return jnp.minimum(r * inv, 448.0).astype(jnp.float8_e4m3fn)

        for j in range(2):
            q_ref[pl.ds(own_a * MC + j * SB, SB), A] = quant(acc_a[j])
            q_ref[pl.ds(own_b * MC + j * SB, SB), B] = quant(acc_b[j])

        copies = []

        def emit(c, cols):
            slot = len(copies) % 2
            if len(copies) >= 2:
                copies[-2].wait()
            rows = pl.ds(c * MC, MC)
            stage[slot, :, :] = (
                q_ref[rows, cols].astype(jnp.float32) * scale
            ).astype(jnp.bfloat16)
            cp = pltpu.make_async_copy(
                stage.at[slot],
                out_ref.at[rows, cols],
                out_sems.at[slot],
            )
            cp.start()
            copies.append(cp)

        def mk_ag(c, cols, snd_sems, rcv_sems, dst, h, j):
            rows = pl.ds(c * MC + j * SB, SB)
            return pltpu.make_async_remote_copy(
                src_ref=q_ref.at[rows, cols],
                dst_ref=q_ref.at[rows, cols],
                send_sem=snd_sems.at[h, j], recv_sem=rcv_sems.at[h, j],
                device_id=(dst,), device_id_type=pl.DeviceIdType.MESH,
            )

        rg_a = {}
        rg_b = {}
        for h in range(N_DEV - 1):
            c_a = mod(my + 1 + h)
            c_b = mod(my - 1 - h)
            for j in range(2):
                if h > 0:
                    rg_a[(h - 1, j)].wait_recv()
                    rg_b[(h - 1, j)].wait_recv()
                rg_a[(h, j)] = mk_ag(
                    c_a, A, ag_snd_a_sems, ag_rcv_a_sems, left, h, j
                )
                rg_b[(h, j)] = mk_ag(
                    c_b, B, ag_snd_b_sems, ag_rcv_b_sems, right, h, j
                )
                rg_a[(h, j)].start()
                rg_b[(h, j)].start()
            emit(c_a, A)
            emit(c_b, B)
        for j in range(2):
            rg_a[(N_DEV - 2, j)].wait_recv()
            rg_b[(N_DEV - 2, j)].wait_recv()
        emit(my, A)
        emit(my, B)
        for rd in list(rg_a.values()) + list(rg_b.values()):
            rd.wait_send()
        copies[-2].wait()
        copies[-1].wait()

    return pl.pallas_call(
        body,
        out_shape=jax.ShapeDtypeStruct((M, N), jnp.bfloat16),
        in_specs=[
            pl.BlockSpec(memory_space=pltpu.VMEM),
            pl.BlockSpec(memory_space=pltpu.VMEM),
        ],
        out_specs=pl.BlockSpec(memory_space=pl.ANY),
        scratch_shapes=[
            pltpu.VMEM((M, N), jnp.float8_e4m3fn),
            pltpu.VMEM((N_DEV - 1, MC, NH), jnp.bfloat16),
            pltpu.VMEM((N_DEV - 1, MC, NH), jnp.bfloat16),
            pltpu.VMEM((MC, NH), jnp.bfloat16),
            pltpu.VMEM((MC, NH), jnp.bfloat16),
            pltpu.VMEM((2, MC, NH), jnp.bfloat16),
            pltpu.VMEM((N_DEV, 8, 128), jnp.float32),
            pltpu.SemaphoreType.DMA((N_DEV - 1, 2)),
            pltpu.SemaphoreType.DMA((N_DEV - 1, 2)),
            pltpu.SemaphoreType.DMA((2,)),
            pltpu.SemaphoreType.DMA((2,)),
            pltpu.SemaphoreType.DMA((N_DEV - 1, 2)),
            pltpu.SemaphoreType.DMA((N_DEV - 1, 2)),
            pltpu.SemaphoreType.DMA((N_DEV - 1, 2)),
            pltpu.SemaphoreType.DMA((N_DEV - 1, 2)),
            pltpu.SemaphoreType.DMA((3,)),
            pltpu.SemaphoreType.DMA((3,)),
            pltpu.SemaphoreType.DMA((2,)),
        ],
        compiler_params=pltpu.CompilerParams(
            collective_id=0,
            vmem_limit_bytes=63 * 1024 * 1024,
        ),
    )(x, w)


# device time: 139741 ns/iter; 2.5774x vs baseline; 1.0001x over previous
import jax
import jax.numpy as jnp
from jax import lax
from jax.experimental import pallas as pl
from jax.experimental.pallas import tpu as pltpu

N_DEV = 4


def kernel(x, w_mat):
    M, _ = x.shape
    _, N = w_mat.shape
    MC = M // N_DEV
    NH = N // 2
    SC = 512
    n_sub = M // SC

    w = w_mat

    def body(x_ref, w_ref, out_ref, q_ref,
             rsr_a, rsr_b, snd_a, snd_b, stage, amax_buf,
             rsr_a_sems, rsr_b_sems, snd_a_sems, snd_b_sems,
             ag_snd_a_sems, ag_rcv_a_sems, ag_snd_b_sems, ag_rcv_b_sems,
             amax_snd_sems, amax_rcv_sems, out_sems):
        my = lax.axis_index("i")
        left = lax.rem(my + (N_DEV - 1), N_DEV)
        right = lax.rem(my + 1, N_DEV)
        opp = lax.rem(my + 2, N_DEV)

        def mod(c):
            return lax.rem(c + N_DEV, N_DEV)

        barrier_sem = pltpu.get_barrier_semaphore()
        for nbr in (left, right):
            pl.semaphore_signal(
                barrier_sem, inc=1, device_id=(nbr,),
                device_id_type=pl.DeviceIdType.MESH,
            )
        pl.semaphore_wait(barrier_sem, 2)

        def pchunk(c, cols):
            xs = x_ref[pl.ds(c * MC, MC), :].astype(jnp.bfloat16)
            ws = w_ref[:, cols].astype(jnp.bfloat16)
            return lax.dot_general(
                xs, ws, (((1,), (0,)), ((), ())),
                preferred_element_type=jnp.float32,
            )

        A = slice(0, NH)
        B = slice(NH, N)

        SB = MC // 2

        def make_dir(snd, rsr, snd_sems, rsr_sems, dst):
            def mk(s, j):
                rows = pl.ds(j * SB, SB)
                return pltpu.make_async_remote_copy(
                    src_ref=snd.at[rows, :],
                    dst_ref=rsr.at[s, rows, :],
                    send_sem=snd_sems.at[j],
                    recv_sem=rsr_sems.at[s, j],
                    device_id=(dst,),
                    device_id_type=pl.DeviceIdType.MESH,
                )
            return mk

        mk_a = make_dir(snd_a, rsr_a, snd_a_sems, rsr_a_sems, right)
        mk_b = make_dir(snd_b, rsr_b, snd_b_sems, rsr_b_sems, left)
        rd_a = {}
        rd_b = {}

        for j in range(2):
            xs = x_ref[pl.ds(my * MC + j * SB, SB), :].astype(jnp.bfloat16)
            pj = lax.dot_general(
                xs, w_ref[...].astype(jnp.bfloat16),
                (((1,), (0,)), ((), ())),
                preferred_element_type=jnp.float32,
            )
            rows = pl.ds(j * SB, SB)
            snd_a[rows, :] = pj[:, A].astype(jnp.bfloat16)
            snd_b[rows, :] = pj[:, B].astype(jnp.bfloat16)
            rd_a[(0, j)] = mk_a(0, j)
            rd_b[(0, j)] = mk_b(0, j)
            rd_a[(0, j)].start()
            rd_b[(0, j)].start()

        pa = pchunk(mod(my - 1), A)
        pb = pchunk(mod(my + 1), B)
        for j in range(2):
            rows = pl.ds(j * SB, SB)
            lo, hi = j * SB, (j + 1) * SB
            rd_a[(0, j)].wait_recv()
            rd_a[(0, j)].wait_send()
            snd_a[rows, :] = (
                pa[lo:hi] + rsr_a[0, rows, :].astype(jnp.float32)
            ).astype(jnp.bfloat16)
            rd_a[(1, j)] = mk_a(1, j)
            rd_a[(1, j)].start()
            rd_b[(0, j)].wait_recv()
            rd_b[(0, j)].wait_send()
            snd_b[rows, :] = (
                pb[lo:hi] + rsr_b[0, rows, :].astype(jnp.float32)
            ).astype(jnp.bfloat16)
            rd_b[(1, j)] = mk_b(1, j)
            rd_b[(1, j)].start()

        p2 = pchunk(mod(my + 2), slice(0, N))
        for j in range(2):
            rows = pl.ds(j * SB, SB)
            lo, hi = j * SB, (j + 1) * SB
            rd_a[(1, j)].wait_recv()
            rd_a[(1, j)].wait_send()
            snd_a[rows, :] = (
                p2[lo:hi, A] + rsr_a[1, rows, :].astype(jnp.float32)
            ).astype(jnp.bfloat16)
            rd_a[(2, j)] = mk_a(2, j)
            rd_a[(2, j)].start()
            rd_b[(1, j)].wait_recv()
            rd_b[(1, j)].wait_send()
            snd_b[rows, :] = (
                p2[lo:hi, B] + rsr_b[1, rows, :].astype(jnp.float32)
            ).astype(jnp.bfloat16)
            rd_b[(2, j)] = mk_b(2, j)
            rd_b[(2, j)].start()

        pa_own = pchunk(mod(my + 1), A)
        pb_own = pchunk(mod(my - 1), B)
        acc_a = [None, None]
        acc_b = [None, None]
        amax_local = 1e-30
        for j in range(2):
            rows = pl.ds(j * SB, SB)
            lo, hi = j * SB, (j + 1) * SB
            rd_a[(2, j)].wait_recv()
            acc_a[j] = pa_own[lo:hi] + rsr_a[2, rows, :].astype(jnp.float32)
            amax_local = jnp.maximum(amax_local, jnp.max(acc_a[j]))
            rd_b[(2, j)].wait_recv()
            acc_b[j] = pb_own[lo:hi] + rsr_b[2, rows, :].astype(jnp.float32)
            amax_local = jnp.maximum(amax_local, jnp.max(acc_b[j]))
        for j in range(2):
            rd_a[(2, j)].wait_send()
            rd_b[(2, j)].wait_send()
        own_a = mod(my + 1)
        own_b = mod(my - 1)

        amax_buf[pl.ds(my, 1), :, :] = jnp.full(
            (1, 8, 128), amax_local, jnp.float32
        )
        peers = (left, right, opp)
        excs = []
        for k, peer in enumerate(peers):
            exc = pltpu.make_async_remote_copy(
                src_ref=amax_buf.at[pl.ds(my, 1)],
                dst_ref=amax_buf.at[pl.ds(my, 1)],
                send_sem=amax_snd_sems.at[k], recv_sem=amax_rcv_sems.at[k],
                device_id=(peer,), device_id_type=pl.DeviceIdType.MESH,
            )
            exc.start()
            excs.append(exc)
        for j in range(2):
            acc_a[j] = jnp.maximum(acc_a[j], 0.0)
            acc_b[j] = jnp.maximum(acc_b[j], 0.0)
        for exc in excs:
            exc.wait()
        amax = jnp.max(amax_buf[...])
        scale = amax / 448.0
        inv = 448.0 / amax

        def quant(r):
            return jnp.minimum(r * inv, 448.0).astype(jnp.float8_e4m3fn)

        for j in range(2):
            q_ref[pl.ds(own_a * MC + j * SB, SB), A] = quant(acc_a[j])
            q_ref[pl.ds(own_b * MC + j * SB, SB), B] = quant(acc_b[j])

        copies = []

        def emit(c, cols):
            slot = len(copies) % 2
            if len(copies) >= 2:
                copies[-2].wait()
            rows = pl.ds(c * MC, MC)
            stage[slot, :, :] = (
                q_ref[rows, cols].astype(jnp.float32) * scale
            ).astype(jnp.bfloat16)
            cp = pltpu.make_async_copy(
                stage.at[slot],
                out_ref.at[rows, cols],
                out_sems.at[slot],
            )
            cp.start()
            copies.append(cp)

        def mk_ag(c, cols, snd_sems, rcv_sems, dst, h, j):
            rows = pl.ds(c * MC + j * SB, SB)
            return pltpu.make_async_remote_copy(
                src_ref=q_ref.at[rows, cols],
                dst_ref=q_ref.at[rows, cols],
                send_sem=snd_sems.at[h, j], recv_sem=rcv_sems.at[h, j],
                device_id=(dst,), device_id_type=pl.DeviceIdType.MESH,
            )

        rg_a = {}
        rg_b = {}
        for h in range(N_DEV - 1):
            c_a = mod(my + 1 + h)
            c_b = mod(my - 1 - h)
            for j in range(2):
                if h > 0:
                    rg_a[(h - 1, j)].wait_recv()
                    rg_b[(h - 1, j)].wait_recv()
                rg_a[(h, j)] = mk_ag(
                    c_a, A, ag_snd_a_sems, ag_rcv_a_sems, left, h, j
                )
                rg_b[(h, j)] = mk_ag(
                    c_b, B, ag_snd_b_sems, ag_rcv_b_sems, right, h, j
                )
                rg_a[(h, j)].start()
                rg_b[(h, j)].start()
            emit(c_a, A)
            emit(c_b, B)
        for j in range(2):
            rg_a[(N_DEV - 2, j)].wait_recv()
            rg_b[(N_DEV - 2, j)].wait_recv()
        emit(my, A)
        emit(my, B)
        for rd in list(rg_a.values()) + list(rg_b.values()):
            rd.wait_send()
        copies[-2].wait()
        copies[-1].wait()

    return pl.pallas_call(
        body,
        out_shape=jax.ShapeDtypeStruct((M, N), jnp.bfloat16),
        in_specs=[
            pl.BlockSpec(memory_space=pltpu.VMEM),
            pl.BlockSpec(memory_space=pltpu.VMEM),
        ],
        out_specs=pl.BlockSpec(memory_space=pl.ANY),
        scratch_shapes=[
            pltpu.VMEM((M, N), jnp.float8_e4m3fn),
            pltpu.VMEM((N_DEV - 1, MC, NH), jnp.bfloat16),
            pltpu.VMEM((N_DEV - 1, MC, NH), jnp.bfloat16),
            pltpu.VMEM((MC, NH), jnp.bfloat16),
            pltpu.VMEM((MC, NH), jnp.bfloat16),
            pltpu.VMEM((2, MC, NH), jnp.bfloat16),
            pltpu.VMEM((N_DEV, 8, 128), jnp.float32),
            pltpu.SemaphoreType.DMA((N_DEV - 1, 2)),
            pltpu.SemaphoreType.DMA((N_DEV - 1, 2)),
            pltpu.SemaphoreType.DMA((2,)),
            pltpu.SemaphoreType.DMA((2,)),
            pltpu.SemaphoreType.DMA((N_DEV - 1, 2)),
            pltpu.SemaphoreType.DMA((N_DEV - 1, 2)),
            pltpu.SemaphoreType.DMA((N_DEV - 1, 2)),
            pltpu.SemaphoreType.DMA((N_DEV - 1, 2)),
            pltpu.SemaphoreType.DMA((3,)),
            pltpu.SemaphoreType.DMA((3,)),
            pltpu.SemaphoreType.DMA((2,)),
        ],
        compiler_params=pltpu.CompilerParams(
            collective_id=0,
            vmem_limit_bytes=63 * 1024 * 1024,
        ),
    )(x, w)
